# Initial kernel scaffold; baseline (speedup 1.0000x reference)
import jax
import jax.numpy as jnp
from jax.experimental import pallas as pl


def kernel(trnn1_w_ih, trnn1_b, trnn1_w_hh, trnn2_w_ih, trnn2_b, trnn2_w_hh, vrnn1_w_ih, vrnn1_b, vrnn1_w_hh, vrnn2_w_ih, vrnn2_b, vrnn2_w_hh, arnn1_w_ih, arnn1_b, arnn1_w_hh, arnn2_w_ih, arnn2_b, arnn2_w_hh, tln_g, tln_b, vln_g, vln_b, aln_g, aln_b, proj_t_w, proj_t_b, proj_t_ln_g, proj_t_ln_b, proj_v_w, proj_v_b, proj_v_ln_g, proj_v_ln_b, proj_a_w, proj_a_b, proj_a_ln_g, proj_a_ln_b, priv_t_w, priv_t_b, priv_v_w, priv_v_b, priv_a_w, priv_a_b, shared_w, shared_b, spd_w, spd_b, fusion_w, fusion_b, tx_in_w, tx_in_b, tx_out_w, tx_out_b, tx_ff1_w, tx_ff1_b, tx_ff2_w, tx_ff2_b, tx_ln1_g, tx_ln1_b, tx_ln2_g, tx_ln2_b, visual, acoustic, sentences):
    raise NotImplementedError("write your pallas kernel here")


# speedup vs baseline: 1.3708x; 1.3708x over previous
import functools
import math

import jax
import jax.numpy as jnp
from jax.experimental import pallas as pl
from jax.experimental.pallas import tpu as pltpu

_EPS = 1e-5
_VMEM = 64 * 1024 * 1024


def _lstm_cell(z, c_prev, hp):
    i = jax.nn.sigmoid(z[:, 0 * hp:1 * hp])
    f = jax.nn.sigmoid(z[:, 1 * hp:2 * hp])
    g = jnp.tanh(z[:, 2 * hp:3 * hp])
    o = jax.nn.sigmoid(z[:, 3 * hp:4 * hp])
    c = f * c_prev + i * g
    return o * jnp.tanh(c), c


def _mod_body(x_ref, w1_ref, b1_ref, whh1_ref, w2_ref, b2_ref, lng_ref,
              lnb_ref, whh2_ref, out_ref, g_scr, h1_scr, h_scr, c_scr,
              *, T, B, H, Hp):
    G4 = 4 * Hp
    G8 = 8 * Hp
    W2 = 2 * Hp

    g_scr[:, 0:G8] = (
        jnp.dot(x_ref[...], w1_ref[...], preferred_element_type=jnp.float32)
        + b1_ref[...]
    )

    def run_layer(whh_ref, store_h):
        h_scr[:, 0:W2] = jnp.zeros((B, W2), jnp.float32)
        c_scr[:, 0:W2] = jnp.zeros((B, W2), jnp.float32)
        for t in range(T):
            gf = g_scr[t * B:(t + 1) * B, 0:G4]
            gb = g_scr[(T - 1 - t) * B:(T - t) * B, G4:G8]
            hf = h_scr[:, 0:Hp]
            hb = h_scr[:, Hp:W2]
            zf = gf + jnp.dot(hf, whh_ref[0],
                              preferred_element_type=jnp.float32)
            zb = gb + jnp.dot(hb, whh_ref[1],
                              preferred_element_type=jnp.float32)
            hf, cf = _lstm_cell(zf, c_scr[:, 0:Hp], Hp)
            hb, cb = _lstm_cell(zb, c_scr[:, Hp:W2], Hp)
            h_scr[:, 0:Hp] = hf
            h_scr[:, Hp:W2] = hb
            c_scr[:, 0:Hp] = cf
            c_scr[:, Hp:W2] = cb
            if store_h:
                h1_scr[t * B:(t + 1) * B, 0:Hp] = hf
                h1_scr[(T - 1 - t) * B:(T - t) * B, Hp:W2] = hb

    run_layer(whh1_ref, store_h=True)
    out_ref[0, :, 0 * Hp:1 * Hp] = h_scr[:, 0:Hp]
    out_ref[0, :, 2 * Hp:3 * Hp] = h_scr[:, Hp:W2]

    x1 = h1_scr[:, 0:W2]
    inv_n = 1.0 / (2 * H)
    mu = jnp.sum(x1, axis=1, keepdims=True) * inv_n
    ex2 = jnp.sum(x1 * x1, axis=1, keepdims=True) * inv_n
    xn = (x1 - mu) * jax.lax.rsqrt(ex2 - mu * mu + _EPS) * lng_ref[...] \
        + lnb_ref[...]
    g_scr[:, 0:G8] = (
        jnp.dot(xn, w2_ref[...], preferred_element_type=jnp.float32)
        + b2_ref[...]
    )
    run_layer(whh2_ref, store_h=False)
    out_ref[0, :, 1 * Hp:2 * Hp] = h_scr[:, 0:Hp]
    out_ref[0, :, 3 * Hp:4 * Hp] = h_scr[:, Hp:W2]


def _feat_kernel(*refs, T, B, dims):
    (xt, w1t, b1t, whh1t, w2t, b2t, lngt, lnbt, whh2t,
     xv, w1v, b1v, whh1v, w2v, b2v, lngv, lnbv, whh2v,
     xa, w1a, b1a, whh1a, w2a, b2a, lnga, lnba, whh2a,
     ut_ref, uv_ref, ua_ref, g_scr, h1_scr, h_scr, c_scr) = refs
    (Ht, Hpt), (Hv, Hpv), (Ha, Hpa) = dims
    pid = pl.program_id(0)

    @pl.when(pid == 0)
    def _():
        _mod_body(xt, w1t, b1t, whh1t, w2t, b2t, lngt, lnbt, whh2t, ut_ref,
                  g_scr, h1_scr, h_scr, c_scr, T=T, B=B, H=Ht, Hp=Hpt)

    @pl.when(pid == 1)
    def _():
        _mod_body(xv, w1v, b1v, whh1v, w2v, b2v, lngv, lnbv, whh2v, uv_ref,
                  g_scr, h1_scr, h_scr, c_scr, T=T, B=B, H=Hv, Hp=Hpv)
        _mod_body(xa, w1a, b1a, whh1a, w2a, b2a, lnga, lnba, whh2a, ua_ref,
                  g_scr, h1_scr, h_scr, c_scr, T=T, B=B, H=Ha, Hp=Hpa)


def _head_kernel(*refs, nhead):
    (ut, uv, ua,
     pwt, pbt, pgt, ptt, pwv, pbv, pgv, ptv, pwa, pba, pga, pta,
     qtw, qtb, qvw, qvb, qaw, qab, shw, shb, sdw, sdb,
     inw, inb, ouw, oub, l1g, l1b, f1w, f1b, f2w, f2b, l2g, l2b,
     fw, fb, o_ref, st_ref, sv_ref, sa_ref, ss_ref) = refs

    E = shw.shape[0]
    B = ut.shape[0]
    S = 6
    SB = S * B
    dh = E // nhead
    scale = 1.0 / math.sqrt(dh)

    def ln(x, g, b):
        mu = jnp.mean(x, axis=-1, keepdims=True)
        xc = x - mu
        var = jnp.mean(xc * xc, axis=-1, keepdims=True)
        return xc * jax.lax.rsqrt(var + _EPS) * g[...] + b[...]

    def lin(x, w, b):
        return jnp.dot(x, w[...], preferred_element_type=jnp.float32) + b[...]

    t = ln(jnp.maximum(lin(ut[...], pwt, pbt), 0.0), pgt, ptt)
    v = ln(jnp.maximum(lin(uv[...], pwv, pbv), 0.0), pgv, ptv)
    a = ln(jnp.maximum(lin(ua[...], pwa, pba), 0.0), pga, pta)

    p_t = jax.nn.sigmoid(lin(t, qtw, qtb))
    p_v = jax.nn.sigmoid(lin(v, qvw, qvb))
    p_a = jax.nn.sigmoid(lin(a, qaw, qab))
    s_t = jax.nn.sigmoid(lin(t, shw, shb))
    s_v = jax.nn.sigmoid(lin(v, shw, shb))
    s_a = jax.nn.sigmoid(lin(a, shw, shb))

    st_ref[...] = lin(p_t, sdw, sdb)
    sv_ref[...] = lin(p_v, sdw, sdb)
    sa_ref[...] = lin(p_a, sdw, sdb)
    ss_ref[...] = lin((s_t + s_v + s_a) / 3.0, sdw, sdb)

    h = jnp.concatenate([p_t, p_v, p_a, s_t, s_v, s_a], axis=0)

    qkv = lin(h, inw, inb)
    q, k, vv = qkv[:, :E], qkv[:, E:2 * E], qkv[:, 2 * E:]
    ri = jax.lax.broadcasted_iota(jnp.int32, (SB, SB), 0)
    rj = jax.lax.broadcasted_iota(jnp.int32, (SB, SB), 1)
    same = (ri % B) == (rj % B)

    attn = jnp.zeros((SB, E), jnp.float32)
    for hd in range(nhead):
        cs = slice(hd * dh, (hd + 1) * dh)
        sc = jax.lax.dot_general(
            q[:, cs], k[:, cs], dimension_numbers=(((1,), (1,)), ((), ())),
            preferred_element_type=jnp.float32) * scale
        sc = jnp.where(same, sc, -1e30)
        m = jnp.max(sc, axis=-1, keepdims=True)
        p = jnp.exp(sc - m)
        p = p / jnp.sum(p, axis=-1, keepdims=True)
        hv = jnp.dot(p, vv[:, cs], preferred_element_type=jnp.float32)
        attn = attn + jnp.dot(hv, ouw[cs, :],
                              preferred_element_type=jnp.float32)

    x = ln(h + attn + oub[...], l1g, l1b)
    x = ln(x + lin(jnp.maximum(lin(x, f1w, f1b), 0.0), f2w, f2b), l2g, l2b)

    o = jnp.zeros((B, fw.shape[1]), jnp.float32)
    for s in range(S):
        o = o + jnp.dot(x[s * B:(s + 1) * B, :], fw[s * E:(s + 1) * E, :],
                        preferred_element_type=jnp.float32)
    o_ref[...] = o + fb[...]


def _pad_rows(w, H, Hp):
    if H == Hp:
        return w
    z = jnp.zeros((Hp - H, w.shape[1]), w.dtype)
    return jnp.concatenate([w[:H], z, w[H:], z], axis=0)


def _pad_vec(g, H, Hp):
    if H == Hp:
        return g.reshape(1, -1)
    z = jnp.zeros((Hp - H,), g.dtype)
    return jnp.concatenate([g[:H], z, g[H:], z]).reshape(1, -1)


def _pad_proj(w, H, Hp):
    if H == Hp:
        return w
    z = jnp.zeros((Hp - H, w.shape[1]), w.dtype)
    return jnp.concatenate(
        [w[0 * H:1 * H], z, w[1 * H:2 * H], z,
         w[2 * H:3 * H], z, w[3 * H:4 * H], z], axis=0)


def kernel(trnn1_w_ih, trnn1_b, trnn1_w_hh, trnn2_w_ih, trnn2_b, trnn2_w_hh,
           vrnn1_w_ih, vrnn1_b, vrnn1_w_hh, vrnn2_w_ih, vrnn2_b, vrnn2_w_hh,
           arnn1_w_ih, arnn1_b, arnn1_w_hh, arnn2_w_ih, arnn2_b, arnn2_w_hh,
           tln_g, tln_b, vln_g, vln_b, aln_g, aln_b,
           proj_t_w, proj_t_b, proj_t_ln_g, proj_t_ln_b,
           proj_v_w, proj_v_b, proj_v_ln_g, proj_v_ln_b,
           proj_a_w, proj_a_b, proj_a_ln_g, proj_a_ln_b,
           priv_t_w, priv_t_b, priv_v_w, priv_v_b, priv_a_w, priv_a_b,
           shared_w, shared_b, spd_w, spd_b, fusion_w, fusion_b,
           tx_in_w, tx_in_b, tx_out_w, tx_out_b,
           tx_ff1_w, tx_ff1_b, tx_ff2_w, tx_ff2_b,
           tx_ln1_g, tx_ln1_b, tx_ln2_g, tx_ln2_b,
           visual, acoustic, sentences):
    B, T, Ht = sentences.shape
    Hv = visual.shape[2]
    Ha = acoustic.shape[2]
    Hpt = trnn1_w_hh.shape[1]
    Hpv = vrnn1_w_hh.shape[1]
    Hpa = arnn1_w_hh.shape[1]

    xt = jnp.transpose(sentences, (1, 0, 2)).reshape(T * B, Ht)
    xv = jnp.transpose(visual, (1, 0, 2)).reshape(T * B, Hv)
    xa = jnp.transpose(acoustic, (1, 0, 2)).reshape(T * B, Ha)

    gw = 8 * max(Hpt, Hpv, Hpa)
    hw = 2 * max(Hpt, Hpv, Hpa)

    r = lambda z: z.reshape(1, -1)
    feat_in = (
        xt, trnn1_w_ih, r(trnn1_b), trnn1_w_hh,
        _pad_rows(trnn2_w_ih, Ht, Hpt), r(trnn2_b),
        _pad_vec(tln_g, Ht, Hpt), _pad_vec(tln_b, Ht, Hpt), trnn2_w_hh,
        xv, vrnn1_w_ih, r(vrnn1_b), vrnn1_w_hh,
        _pad_rows(vrnn2_w_ih, Hv, Hpv), r(vrnn2_b),
        _pad_vec(vln_g, Hv, Hpv), _pad_vec(vln_b, Hv, Hpv), vrnn2_w_hh,
        xa, arnn1_w_ih, r(arnn1_b), arnn1_w_hh,
        _pad_rows(arnn2_w_ih, Ha, Hpa), r(arnn2_b),
        _pad_vec(aln_g, Ha, Hpa), _pad_vec(aln_b, Ha, Hpa), arnn2_w_hh,
    )

    def full(arr):
        zero = (0,) * arr.ndim
        return pl.BlockSpec(arr.shape, lambda i, _z=zero: _z)

    out_specs = [
        pl.BlockSpec((1, B, 4 * Hpt), lambda i: (i, 0, 0)),
        pl.BlockSpec((1, B, 4 * Hpv), lambda i: (i, 0, 0)),
        pl.BlockSpec((1, B, 4 * Hpa), lambda i: (i, 0, 0)),
    ]
    ut, uv, ua = pl.pallas_call(
        functools.partial(_feat_kernel, T=T, B=B,
                          dims=((Ht, Hpt), (Hv, Hpv), (Ha, Hpa))),
        grid=(2,),
        in_specs=[full(a) for a in feat_in],
        out_specs=out_specs,
        out_shape=[
            jax.ShapeDtypeStruct((2, B, 4 * Hpt), jnp.float32),
            jax.ShapeDtypeStruct((2, B, 4 * Hpv), jnp.float32),
            jax.ShapeDtypeStruct((2, B, 4 * Hpa), jnp.float32),
        ],
        scratch_shapes=[
            pltpu.VMEM((T * B, gw), jnp.float32),
            pltpu.VMEM((T * B, hw), jnp.float32),
            pltpu.VMEM((B, hw), jnp.float32),
            pltpu.VMEM((B, hw), jnp.float32),
        ],
        compiler_params=pltpu.CompilerParams(
            dimension_semantics=("parallel",),
            vmem_limit_bytes=_VMEM,
        ),
    )(*feat_in)

    head_in = (
        ut[0], uv[1], ua[1],
        _pad_proj(proj_t_w, Ht, Hpt), r(proj_t_b),
        r(proj_t_ln_g), r(proj_t_ln_b),
        _pad_proj(proj_v_w, Hv, Hpv), r(proj_v_b),
        r(proj_v_ln_g), r(proj_v_ln_b),
        _pad_proj(proj_a_w, Ha, Hpa), r(proj_a_b),
        r(proj_a_ln_g), r(proj_a_ln_b),
        priv_t_w, r(priv_t_b), priv_v_w, r(priv_v_b), priv_a_w, r(priv_a_b),
        shared_w, r(shared_b), spd_w, r(spd_b),
        tx_in_w, r(tx_in_b), tx_out_w, r(tx_out_b),
        r(tx_ln1_g), r(tx_ln1_b),
        tx_ff1_w, r(tx_ff1_b), tx_ff2_w, r(tx_ff2_b),
        r(tx_ln2_g), r(tx_ln2_b),
        fusion_w, r(fusion_b),
    )
    E = shared_w.shape[0]
    o, spt, spv, spa, sps = pl.pallas_call(
        functools.partial(_head_kernel, nhead=2),
        out_shape=(
            jax.ShapeDtypeStruct((B, 3 * E), jnp.float32),
            jax.ShapeDtypeStruct((B, 4), jnp.float32),
            jax.ShapeDtypeStruct((B, 4), jnp.float32),
            jax.ShapeDtypeStruct((B, 4), jnp.float32),
            jax.ShapeDtypeStruct((B, 4), jnp.float32),
        ),
        compiler_params=pltpu.CompilerParams(vmem_limit_bytes=_VMEM),
    )(*head_in)
    aux = {"sp_p_t": spt, "sp_p_v": spv, "sp_p_a": spa, "sp_s": sps}
    return o, aux


# speedup vs baseline: 1.4764x; 1.0770x over previous
import functools
import math

import jax
import jax.numpy as jnp
from jax.experimental import pallas as pl
from jax.experimental.pallas import tpu as pltpu

_EPS = 1e-5
_VMEM = 64 * 1024 * 1024


def _lstm_cell(z, c_prev, hp):
    i = jax.nn.sigmoid(z[:, 0 * hp:1 * hp])
    f = jax.nn.sigmoid(z[:, 1 * hp:2 * hp])
    g = jnp.tanh(z[:, 2 * hp:3 * hp])
    o = jax.nn.sigmoid(z[:, 3 * hp:4 * hp])
    c = f * c_prev + i * g
    return o * jnp.tanh(c), c


def _feat_kernel(*refs, T, B, dims):
    (xt, w1t, b1t, whh1t, w2t, b2t, lngt, lnbt, whh2t,
     xv, w1v, b1v, whh1v, w2v, b2v, lngv, lnbv, whh2v,
     xa, w1a, b1a, whh1a, w2a, b2a, lnga, lnba, whh2a,
     ut_ref, uv_ref, ua_ref,
     gt, gv, ga, h1t, h1v, h1a,
     hst, cst, hsv, csv, hsa, csa) = refs

    mods = []
    for (H, Hp), g_scr, h1, hs, cs, whh1, whh2, x, w1, b1, w2, b2, lng, \
            lnb, out in (
            (dims[0], gt, h1t, hst, cst, whh1t, whh2t, xt, w1t, b1t, w2t,
             b2t, lngt, lnbt, ut_ref),
            (dims[1], gv, h1v, hsv, csv, whh1v, whh2v, xv, w1v, b1v, w2v,
             b2v, lngv, lnbv, uv_ref),
            (dims[2], ga, h1a, hsa, csa, whh1a, whh2a, xa, w1a, b1a, w2a,
             b2a, lnga, lnba, ua_ref)):
        mods.append(dict(H=H, Hp=Hp, g=g_scr, h1=h1, hs=hs, cs=cs,
                         whh1=whh1, whh2=whh2, x=x, w1=w1, b1=b1, w2=w2,
                         b2=b2, lng=lng, lnb=lnb, out=out))

    for m in mods:
        m["g"][...] = (
            jnp.dot(m["x"][...], m["w1"][...],
                    preferred_element_type=jnp.float32) + m["b1"][...]
        )

    def run_layer(whh_key, store_h):
        for m in mods:
            W2 = 2 * m["Hp"]
            m["hs"][...] = jnp.zeros((B, W2), jnp.float32)
            m["cs"][...] = jnp.zeros((B, W2), jnp.float32)
        for t in range(T):
            zs = []
            for m in mods:
                Hp, G4 = m["Hp"], 4 * m["Hp"]
                zf = m["g"][t * B:(t + 1) * B, 0:G4] + jnp.dot(
                    m["hs"][:, 0:Hp], m[whh_key][0],
                    preferred_element_type=jnp.float32)
                zb = m["g"][(T - 1 - t) * B:(T - t) * B, G4:2 * G4] + \
                    jnp.dot(m["hs"][:, Hp:2 * Hp], m[whh_key][1],
                            preferred_element_type=jnp.float32)
                zs.append((zf, zb))
            for m, (zf, zb) in zip(mods, zs):
                Hp = m["Hp"]
                hf, cf = _lstm_cell(zf, m["cs"][:, 0:Hp], Hp)
                hb, cb = _lstm_cell(zb, m["cs"][:, Hp:2 * Hp], Hp)
                m["hs"][:, 0:Hp] = hf
                m["hs"][:, Hp:2 * Hp] = hb
                m["cs"][:, 0:Hp] = cf
                m["cs"][:, Hp:2 * Hp] = cb
                if store_h:
                    m["h1"][t * B:(t + 1) * B, 0:Hp] = hf
                    m["h1"][(T - 1 - t) * B:(T - t) * B, Hp:2 * Hp] = hb

    run_layer("whh1", store_h=True)
    for m in mods:
        Hp = m["Hp"]
        m["out"][:, 0 * Hp:1 * Hp] = m["hs"][:, 0:Hp]
        m["out"][:, 2 * Hp:3 * Hp] = m["hs"][:, Hp:2 * Hp]

    for m in mods:
        x1 = m["h1"][...]
        inv_n = 1.0 / (2 * m["H"])
        mu = jnp.sum(x1, axis=1, keepdims=True) * inv_n
        ex2 = jnp.sum(x1 * x1, axis=1, keepdims=True) * inv_n
        xn = (x1 - mu) * jax.lax.rsqrt(ex2 - mu * mu + _EPS) * \
            m["lng"][...] + m["lnb"][...]
        m["g"][...] = (
            jnp.dot(xn, m["w2"][...], preferred_element_type=jnp.float32)
            + m["b2"][...]
        )

    run_layer("whh2", store_h=False)
    for m in mods:
        Hp = m["Hp"]
        m["out"][:, 1 * Hp:2 * Hp] = m["hs"][:, 0:Hp]
        m["out"][:, 3 * Hp:4 * Hp] = m["hs"][:, Hp:2 * Hp]


def _head_kernel(*refs, nhead):
    (ut, uv, ua,
     pwt, pbt, pgt, ptt, pwv, pbv, pgv, ptv, pwa, pba, pga, pta,
     qtw, qtb, qvw, qvb, qaw, qab, shw, shb, sdw, sdb,
     inw, inb, ouw, oub, l1g, l1b, f1w, f1b, f2w, f2b, l2g, l2b,
     fw, fb, o_ref, st_ref, sv_ref, sa_ref, ss_ref) = refs

    E = shw.shape[0]
    B = ut.shape[0]
    S = 6
    SB = S * B
    dh = E // nhead
    scale = 1.0 / math.sqrt(dh)

    def ln(x, g, b):
        mu = jnp.mean(x, axis=-1, keepdims=True)
        xc = x - mu
        var = jnp.mean(xc * xc, axis=-1, keepdims=True)
        return xc * jax.lax.rsqrt(var + _EPS) * g[...] + b[...]

    def lin(x, w, b):
        return jnp.dot(x, w[...], preferred_element_type=jnp.float32) + b[...]

    t = ln(jnp.maximum(lin(ut[...], pwt, pbt), 0.0), pgt, ptt)
    v = ln(jnp.maximum(lin(uv[...], pwv, pbv), 0.0), pgv, ptv)
    a = ln(jnp.maximum(lin(ua[...], pwa, pba), 0.0), pga, pta)

    p_t = jax.nn.sigmoid(lin(t, qtw, qtb))
    p_v = jax.nn.sigmoid(lin(v, qvw, qvb))
    p_a = jax.nn.sigmoid(lin(a, qaw, qab))
    s_t = jax.nn.sigmoid(lin(t, shw, shb))
    s_v = jax.nn.sigmoid(lin(v, shw, shb))
    s_a = jax.nn.sigmoid(lin(a, shw, shb))

    st_ref[...] = lin(p_t, sdw, sdb)
    sv_ref[...] = lin(p_v, sdw, sdb)
    sa_ref[...] = lin(p_a, sdw, sdb)
    ss_ref[...] = lin((s_t + s_v + s_a) / 3.0, sdw, sdb)

    h = jnp.concatenate([p_t, p_v, p_a, s_t, s_v, s_a], axis=0)

    qkv = lin(h, inw, inb)
    q, k, vv = qkv[:, :E], qkv[:, E:2 * E], qkv[:, 2 * E:]
    ri = jax.lax.broadcasted_iota(jnp.int32, (SB, SB), 0)
    rj = jax.lax.broadcasted_iota(jnp.int32, (SB, SB), 1)
    same = (ri % B) == (rj % B)

    attn = jnp.zeros((SB, E), jnp.float32)
    for hd in range(nhead):
        cs = slice(hd * dh, (hd + 1) * dh)
        sc = jax.lax.dot_general(
            q[:, cs], k[:, cs], dimension_numbers=(((1,), (1,)), ((), ())),
            preferred_element_type=jnp.float32) * scale
        sc = jnp.where(same, sc, -1e30)
        m = jnp.max(sc, axis=-1, keepdims=True)
        p = jnp.exp(sc - m)
        p = p / jnp.sum(p, axis=-1, keepdims=True)
        hv = jnp.dot(p, vv[:, cs], preferred_element_type=jnp.float32)
        attn = attn + jnp.dot(hv, ouw[cs, :],
                              preferred_element_type=jnp.float32)

    x = ln(h + attn + oub[...], l1g, l1b)
    x = ln(x + lin(jnp.maximum(lin(x, f1w, f1b), 0.0), f2w, f2b), l2g, l2b)

    o = jnp.zeros((B, fw.shape[1]), jnp.float32)
    for s in range(S):
        o = o + jnp.dot(x[s * B:(s + 1) * B, :], fw[s * E:(s + 1) * E, :],
                        preferred_element_type=jnp.float32)
    o_ref[...] = o + fb[...]


def _pad_rows(w, H, Hp):
    if H == Hp:
        return w
    z = jnp.zeros((Hp - H, w.shape[1]), w.dtype)
    return jnp.concatenate([w[:H], z, w[H:], z], axis=0)


def _pad_vec(g, H, Hp):
    if H == Hp:
        return g.reshape(1, -1)
    z = jnp.zeros((Hp - H,), g.dtype)
    return jnp.concatenate([g[:H], z, g[H:], z]).reshape(1, -1)


def _pad_proj(w, H, Hp):
    if H == Hp:
        return w
    z = jnp.zeros((Hp - H, w.shape[1]), w.dtype)
    return jnp.concatenate(
        [w[0 * H:1 * H], z, w[1 * H:2 * H], z,
         w[2 * H:3 * H], z, w[3 * H:4 * H], z], axis=0)


def kernel(trnn1_w_ih, trnn1_b, trnn1_w_hh, trnn2_w_ih, trnn2_b, trnn2_w_hh,
           vrnn1_w_ih, vrnn1_b, vrnn1_w_hh, vrnn2_w_ih, vrnn2_b, vrnn2_w_hh,
           arnn1_w_ih, arnn1_b, arnn1_w_hh, arnn2_w_ih, arnn2_b, arnn2_w_hh,
           tln_g, tln_b, vln_g, vln_b, aln_g, aln_b,
           proj_t_w, proj_t_b, proj_t_ln_g, proj_t_ln_b,
           proj_v_w, proj_v_b, proj_v_ln_g, proj_v_ln_b,
           proj_a_w, proj_a_b, proj_a_ln_g, proj_a_ln_b,
           priv_t_w, priv_t_b, priv_v_w, priv_v_b, priv_a_w, priv_a_b,
           shared_w, shared_b, spd_w, spd_b, fusion_w, fusion_b,
           tx_in_w, tx_in_b, tx_out_w, tx_out_b,
           tx_ff1_w, tx_ff1_b, tx_ff2_w, tx_ff2_b,
           tx_ln1_g, tx_ln1_b, tx_ln2_g, tx_ln2_b,
           visual, acoustic, sentences):
    B, T, Ht = sentences.shape
    Hv = visual.shape[2]
    Ha = acoustic.shape[2]
    Hpt = trnn1_w_hh.shape[1]
    Hpv = vrnn1_w_hh.shape[1]
    Hpa = arnn1_w_hh.shape[1]

    xt = jnp.transpose(sentences, (1, 0, 2)).reshape(T * B, Ht)
    xv = jnp.transpose(visual, (1, 0, 2)).reshape(T * B, Hv)
    xa = jnp.transpose(acoustic, (1, 0, 2)).reshape(T * B, Ha)

    r = lambda z: z.reshape(1, -1)
    feat_in = (
        xt, trnn1_w_ih, r(trnn1_b), trnn1_w_hh,
        _pad_rows(trnn2_w_ih, Ht, Hpt), r(trnn2_b),
        _pad_vec(tln_g, Ht, Hpt), _pad_vec(tln_b, Ht, Hpt), trnn2_w_hh,
        xv, vrnn1_w_ih, r(vrnn1_b), vrnn1_w_hh,
        _pad_rows(vrnn2_w_ih, Hv, Hpv), r(vrnn2_b),
        _pad_vec(vln_g, Hv, Hpv), _pad_vec(vln_b, Hv, Hpv), vrnn2_w_hh,
        xa, arnn1_w_ih, r(arnn1_b), arnn1_w_hh,
        _pad_rows(arnn2_w_ih, Ha, Hpa), r(arnn2_b),
        _pad_vec(aln_g, Ha, Hpa), _pad_vec(aln_b, Ha, Hpa), arnn2_w_hh,
    )

    ut, uv, ua = pl.pallas_call(
        functools.partial(_feat_kernel, T=T, B=B,
                          dims=((Ht, Hpt), (Hv, Hpv), (Ha, Hpa))),
        out_shape=[
            jax.ShapeDtypeStruct((B, 4 * Hpt), jnp.float32),
            jax.ShapeDtypeStruct((B, 4 * Hpv), jnp.float32),
            jax.ShapeDtypeStruct((B, 4 * Hpa), jnp.float32),
        ],
        scratch_shapes=[
            pltpu.VMEM((T * B, 8 * Hpt), jnp.float32),
            pltpu.VMEM((T * B, 8 * Hpv), jnp.float32),
            pltpu.VMEM((T * B, 8 * Hpa), jnp.float32),
            pltpu.VMEM((T * B, 2 * Hpt), jnp.float32),
            pltpu.VMEM((T * B, 2 * Hpv), jnp.float32),
            pltpu.VMEM((T * B, 2 * Hpa), jnp.float32),
            pltpu.VMEM((B, 2 * Hpt), jnp.float32),
            pltpu.VMEM((B, 2 * Hpt), jnp.float32),
            pltpu.VMEM((B, 2 * Hpv), jnp.float32),
            pltpu.VMEM((B, 2 * Hpv), jnp.float32),
            pltpu.VMEM((B, 2 * Hpa), jnp.float32),
            pltpu.VMEM((B, 2 * Hpa), jnp.float32),
        ],
        compiler_params=pltpu.CompilerParams(vmem_limit_bytes=_VMEM),
    )(*feat_in)

    head_in = (
        ut, uv, ua,
        _pad_proj(proj_t_w, Ht, Hpt), r(proj_t_b),
        r(proj_t_ln_g), r(proj_t_ln_b),
        _pad_proj(proj_v_w, Hv, Hpv), r(proj_v_b),
        r(proj_v_ln_g), r(proj_v_ln_b),
        _pad_proj(proj_a_w, Ha, Hpa), r(proj_a_b),
        r(proj_a_ln_g), r(proj_a_ln_b),
        priv_t_w, r(priv_t_b), priv_v_w, r(priv_v_b), priv_a_w, r(priv_a_b),
        shared_w, r(shared_b), spd_w, r(spd_b),
        tx_in_w, r(tx_in_b), tx_out_w, r(tx_out_b),
        r(tx_ln1_g), r(tx_ln1_b),
        tx_ff1_w, r(tx_ff1_b), tx_ff2_w, r(tx_ff2_b),
        r(tx_ln2_g), r(tx_ln2_b),
        fusion_w, r(fusion_b),
    )
    E = shared_w.shape[0]
    o, spt, spv, spa, sps = pl.pallas_call(
        functools.partial(_head_kernel, nhead=2),
        out_shape=(
            jax.ShapeDtypeStruct((B, 3 * E), jnp.float32),
            jax.ShapeDtypeStruct((B, 4), jnp.float32),
            jax.ShapeDtypeStruct((B, 4), jnp.float32),
            jax.ShapeDtypeStruct((B, 4), jnp.float32),
            jax.ShapeDtypeStruct((B, 4), jnp.float32),
        ),
        compiler_params=pltpu.CompilerParams(vmem_limit_bytes=_VMEM),
    )(*head_in)
    aux = {"sp_p_t": spt, "sp_p_v": spv, "sp_p_a": spa, "sp_s": sps}
    return o, aux


# speedup vs baseline: 2.1663x; 1.4673x over previous
import functools
import math

import jax
import jax.numpy as jnp
from jax.experimental import pallas as pl
from jax.experimental.pallas import tpu as pltpu

_EPS = 1e-5
_VMEM = 64 * 1024 * 1024


def _lstm_cell(z, c_prev, hp):
    i = jax.nn.sigmoid(z[:, 0 * hp:1 * hp])
    f = jax.nn.sigmoid(z[:, 1 * hp:2 * hp])
    g = jnp.tanh(z[:, 2 * hp:3 * hp])
    o = jax.nn.sigmoid(z[:, 3 * hp:4 * hp])
    c = f * c_prev + i * g
    return o * jnp.tanh(c), c


def _feat_kernel(*refs, T, B, dims):
    (xt, w1t, b1t, whh1t, w2t, b2t, lngt, lnbt, whh2t,
     xv, w1v, b1v, whh1v, w2v, b2v, lngv, lnbv, whh2v,
     xa, w1a, b1a, whh1a, w2a, b2a, lnga, lnba, whh2a,
     ut_ref, uv_ref, ua_ref,
     gt, gv, ga, h1t, h1v, h1a,
     hst, cst, hsv, csv, hsa, csa,
     w2tb, w2vp, w2ap) = refs

    mods = []
    for (H, Hp), g_scr, h1, hs, cs, whh1, whh2, x, w1, b1, w2, b2, lng, \
            lnb, out in (
            (dims[0], gt, h1t, hst, cst, whh1t, whh2t, xt, w1t, b1t, w2t,
             b2t, lngt, lnbt, ut_ref),
            (dims[1], gv, h1v, hsv, csv, whh1v, whh2v, xv, w1v, b1v, w2v,
             b2v, lngv, lnbv, uv_ref),
            (dims[2], ga, h1a, hsa, csa, whh1a, whh2a, xa, w1a, b1a, w2a,
             b2a, lnga, lnba, ua_ref)):
        mods.append(dict(H=H, Hp=Hp, g=g_scr, h1=h1, hs=hs, cs=cs,
                         whh1=whh1, whh2=whh2, x=x, w1=w1, b1=b1, w2=w2,
                         b2=b2, lng=lng, lnb=lnb, out=out))

    w2scrs = (w2tb, w2vp, w2ap)
    for scr, m in zip(w2scrs, mods):
        H, Hp = m["H"], m["Hp"]
        N = scr.shape[1]
        if 2 * H >= Hp:
            scr[0:H, :] = m["w2"][H:2 * H, :]
            scr[H:Hp, :] = jnp.zeros((Hp - H, N), jnp.float32)
        else:
            scr[0:H, :] = m["w2"][0:H, :]
            scr[H:Hp, :] = jnp.zeros((Hp - H, N), jnp.float32)
            scr[Hp:Hp + H, :] = m["w2"][H:2 * H, :]
            scr[Hp + H:2 * Hp, :] = jnp.zeros((Hp - H, N), jnp.float32)

    for m in mods:
        m["g"][...] = (
            jnp.dot(m["x"][...], m["w1"][...],
                    preferred_element_type=jnp.float32) + m["b1"][...]
        )

    def run_layer(whh_key, store_h):
        for m in mods:
            W2 = 2 * m["Hp"]
            m["hs"][...] = jnp.zeros((B, W2), jnp.float32)
            m["cs"][...] = jnp.zeros((B, W2), jnp.float32)
        for t in range(T):
            zs = []
            for m in mods:
                Hp, G4 = m["Hp"], 4 * m["Hp"]
                zf = m["g"][t * B:(t + 1) * B, 0:G4] + jnp.dot(
                    m["hs"][:, 0:Hp], m[whh_key][0],
                    preferred_element_type=jnp.float32)
                zb = m["g"][(T - 1 - t) * B:(T - t) * B, G4:2 * G4] + \
                    jnp.dot(m["hs"][:, Hp:2 * Hp], m[whh_key][1],
                            preferred_element_type=jnp.float32)
                zs.append((zf, zb))
            for m, (zf, zb) in zip(mods, zs):
                Hp = m["Hp"]
                hf, cf = _lstm_cell(zf, m["cs"][:, 0:Hp], Hp)
                hb, cb = _lstm_cell(zb, m["cs"][:, Hp:2 * Hp], Hp)
                m["hs"][:, 0:Hp] = hf
                m["hs"][:, Hp:2 * Hp] = hb
                m["cs"][:, 0:Hp] = cf
                m["cs"][:, Hp:2 * Hp] = cb
                if store_h:
                    m["h1"][t * B:(t + 1) * B, 0:Hp] = hf
                    m["h1"][(T - 1 - t) * B:(T - t) * B, Hp:2 * Hp] = hb

    run_layer("whh1", store_h=True)
    for m in mods:
        Hp = m["Hp"]
        m["out"][:, 0 * Hp:1 * Hp] = m["hs"][:, 0:Hp]
        m["out"][:, 2 * Hp:3 * Hp] = m["hs"][:, Hp:2 * Hp]

    for scr, m in zip(w2scrs, mods):
        H, Hp = m["H"], m["Hp"]
        x1 = m["h1"][...]
        inv_n = 1.0 / (2 * H)
        mu = jnp.sum(x1, axis=1, keepdims=True) * inv_n
        ex2 = jnp.sum(x1 * x1, axis=1, keepdims=True) * inv_n
        xn = (x1 - mu) * jax.lax.rsqrt(ex2 - mu * mu + _EPS) * \
            m["lng"][...] + m["lnb"][...]
        if 2 * H >= Hp:
            m["g"][...] = (
                jnp.dot(xn[:, 0:Hp], m["w2"][0:Hp, :],
                        preferred_element_type=jnp.float32)
                + jnp.dot(xn[:, Hp:2 * Hp], scr[...],
                          preferred_element_type=jnp.float32)
                + m["b2"][...]
            )
        else:
            m["g"][...] = (
                jnp.dot(xn, scr[...], preferred_element_type=jnp.float32)
                + m["b2"][...]
            )

    run_layer("whh2", store_h=False)
    for m in mods:
        Hp = m["Hp"]
        m["out"][:, 1 * Hp:2 * Hp] = m["hs"][:, 0:Hp]
        m["out"][:, 3 * Hp:4 * Hp] = m["hs"][:, Hp:2 * Hp]


def _head_kernel(*refs, nhead, dims):
    (ut, uv, ua,
     pwt, pbt, pgt, ptt, pwv, pbv, pgv, ptv, pwa, pba, pga, pta,
     qtw, qtb, qvw, qvb, qaw, qab, shw, shb, sdw, sdb,
     inw, inb, ouw, oub, l1g, l1b, f1w, f1b, f2w, f2b, l2g, l2b,
     fw, fb, o_ref, st_ref, sv_ref, sa_ref, ss_ref,
     pts, pvs, pas) = refs

    E = shw.shape[0]
    B = ut.shape[0]
    S = 6
    SB = S * B
    dh = E // nhead
    scale = 1.0 / math.sqrt(dh)

    for scr, w, (H, Hp) in ((pts, pwt, dims[0]), (pvs, pwv, dims[1]),
                            (pas, pwa, dims[2])):
        if H == Hp:
            scr[...] = w[...]
        else:
            for k in range(4):
                scr[k * Hp:k * Hp + H, :] = w[k * H:(k + 1) * H, :]
                scr[k * Hp + H:(k + 1) * Hp, :] = jnp.zeros(
                    (Hp - H, E), jnp.float32)
    pwt, pwv, pwa = pts, pvs, pas

    def ln(x, g, b):
        mu = jnp.mean(x, axis=-1, keepdims=True)
        xc = x - mu
        var = jnp.mean(xc * xc, axis=-1, keepdims=True)
        return xc * jax.lax.rsqrt(var + _EPS) * g[...] + b[...]

    def lin(x, w, b):
        return jnp.dot(x, w[...], preferred_element_type=jnp.float32) + b[...]

    t = ln(jnp.maximum(lin(ut[...], pwt, pbt), 0.0), pgt, ptt)
    v = ln(jnp.maximum(lin(uv[...], pwv, pbv), 0.0), pgv, ptv)
    a = ln(jnp.maximum(lin(ua[...], pwa, pba), 0.0), pga, pta)

    p_t = jax.nn.sigmoid(lin(t, qtw, qtb))
    p_v = jax.nn.sigmoid(lin(v, qvw, qvb))
    p_a = jax.nn.sigmoid(lin(a, qaw, qab))
    s_t = jax.nn.sigmoid(lin(t, shw, shb))
    s_v = jax.nn.sigmoid(lin(v, shw, shb))
    s_a = jax.nn.sigmoid(lin(a, shw, shb))

    st_ref[...] = lin(p_t, sdw, sdb)
    sv_ref[...] = lin(p_v, sdw, sdb)
    sa_ref[...] = lin(p_a, sdw, sdb)
    ss_ref[...] = lin((s_t + s_v + s_a) / 3.0, sdw, sdb)

    h = jnp.concatenate([p_t, p_v, p_a, s_t, s_v, s_a], axis=0)

    qkv = lin(h, inw, inb)
    q, k, vv = qkv[:, :E], qkv[:, E:2 * E], qkv[:, 2 * E:]
    ri = jax.lax.broadcasted_iota(jnp.int32, (SB, SB), 0)
    rj = jax.lax.broadcasted_iota(jnp.int32, (SB, SB), 1)
    same = (ri % B) == (rj % B)

    attn = jnp.zeros((SB, E), jnp.float32)
    for hd in range(nhead):
        cs = slice(hd * dh, (hd + 1) * dh)
        sc = jax.lax.dot_general(
            q[:, cs], k[:, cs], dimension_numbers=(((1,), (1,)), ((), ())),
            preferred_element_type=jnp.float32) * scale
        sc = jnp.where(same, sc, -1e30)
        m = jnp.max(sc, axis=-1, keepdims=True)
        p = jnp.exp(sc - m)
        p = p / jnp.sum(p, axis=-1, keepdims=True)
        hv = jnp.dot(p, vv[:, cs], preferred_element_type=jnp.float32)
        attn = attn + jnp.dot(hv, ouw[cs, :],
                              preferred_element_type=jnp.float32)

    x = ln(h + attn + oub[...], l1g, l1b)
    x = ln(x + lin(jnp.maximum(lin(x, f1w, f1b), 0.0), f2w, f2b), l2g, l2b)

    o = jnp.zeros((B, fw.shape[1]), jnp.float32)
    for s in range(S):
        o = o + jnp.dot(x[s * B:(s + 1) * B, :], fw[s * E:(s + 1) * E, :],
                        preferred_element_type=jnp.float32)
    o_ref[...] = o + fb[...]


def _pad_vec(g, H, Hp):
    if H == Hp:
        return g.reshape(1, -1)
    z = jnp.zeros((Hp - H,), g.dtype)
    return jnp.concatenate([g[:H], z, g[H:], z]).reshape(1, -1)


def kernel(trnn1_w_ih, trnn1_b, trnn1_w_hh, trnn2_w_ih, trnn2_b, trnn2_w_hh,
           vrnn1_w_ih, vrnn1_b, vrnn1_w_hh, vrnn2_w_ih, vrnn2_b, vrnn2_w_hh,
           arnn1_w_ih, arnn1_b, arnn1_w_hh, arnn2_w_ih, arnn2_b, arnn2_w_hh,
           tln_g, tln_b, vln_g, vln_b, aln_g, aln_b,
           proj_t_w, proj_t_b, proj_t_ln_g, proj_t_ln_b,
           proj_v_w, proj_v_b, proj_v_ln_g, proj_v_ln_b,
           proj_a_w, proj_a_b, proj_a_ln_g, proj_a_ln_b,
           priv_t_w, priv_t_b, priv_v_w, priv_v_b, priv_a_w, priv_a_b,
           shared_w, shared_b, spd_w, spd_b, fusion_w, fusion_b,
           tx_in_w, tx_in_b, tx_out_w, tx_out_b,
           tx_ff1_w, tx_ff1_b, tx_ff2_w, tx_ff2_b,
           tx_ln1_g, tx_ln1_b, tx_ln2_g, tx_ln2_b,
           visual, acoustic, sentences):
    B, T, Ht = sentences.shape
    Hv = visual.shape[2]
    Ha = acoustic.shape[2]
    Hpt = trnn1_w_hh.shape[1]
    Hpv = vrnn1_w_hh.shape[1]
    Hpa = arnn1_w_hh.shape[1]

    xt = jnp.transpose(sentences, (1, 0, 2)).reshape(T * B, Ht)
    xv = jnp.transpose(visual, (1, 0, 2)).reshape(T * B, Hv)
    xa = jnp.transpose(acoustic, (1, 0, 2)).reshape(T * B, Ha)

    r = lambda z: z.reshape(1, -1)
    feat_in = (
        xt, trnn1_w_ih, r(trnn1_b), trnn1_w_hh,
        trnn2_w_ih, r(trnn2_b),
        _pad_vec(tln_g, Ht, Hpt), _pad_vec(tln_b, Ht, Hpt), trnn2_w_hh,
        xv, vrnn1_w_ih, r(vrnn1_b), vrnn1_w_hh,
        vrnn2_w_ih, r(vrnn2_b),
        _pad_vec(vln_g, Hv, Hpv), _pad_vec(vln_b, Hv, Hpv), vrnn2_w_hh,
        xa, arnn1_w_ih, r(arnn1_b), arnn1_w_hh,
        arnn2_w_ih, r(arnn2_b),
        _pad_vec(aln_g, Ha, Hpa), _pad_vec(aln_b, Ha, Hpa), arnn2_w_hh,
    )

    ut, uv, ua = pl.pallas_call(
        functools.partial(_feat_kernel, T=T, B=B,
                          dims=((Ht, Hpt), (Hv, Hpv), (Ha, Hpa))),
        out_shape=[
            jax.ShapeDtypeStruct((B, 4 * Hpt), jnp.float32),
            jax.ShapeDtypeStruct((B, 4 * Hpv), jnp.float32),
            jax.ShapeDtypeStruct((B, 4 * Hpa), jnp.float32),
        ],
        scratch_shapes=[
            pltpu.VMEM((T * B, 8 * Hpt), jnp.float32),
            pltpu.VMEM((T * B, 8 * Hpv), jnp.float32),
            pltpu.VMEM((T * B, 8 * Hpa), jnp.float32),
            pltpu.VMEM((T * B, 2 * Hpt), jnp.float32),
            pltpu.VMEM((T * B, 2 * Hpv), jnp.float32),
            pltpu.VMEM((T * B, 2 * Hpa), jnp.float32),
            pltpu.VMEM((B, 2 * Hpt), jnp.float32),
            pltpu.VMEM((B, 2 * Hpt), jnp.float32),
            pltpu.VMEM((B, 2 * Hpv), jnp.float32),
            pltpu.VMEM((B, 2 * Hpv), jnp.float32),
            pltpu.VMEM((B, 2 * Hpa), jnp.float32),
            pltpu.VMEM((B, 2 * Hpa), jnp.float32),
            pltpu.VMEM((Hpt if 2 * Ht >= Hpt else 2 * Hpt, 8 * Hpt),
                       jnp.float32),
            pltpu.VMEM((Hpv if 2 * Hv >= Hpv else 2 * Hpv, 8 * Hpv),
                       jnp.float32),
            pltpu.VMEM((Hpa if 2 * Ha >= Hpa else 2 * Hpa, 8 * Hpa),
                       jnp.float32),
        ],
        compiler_params=pltpu.CompilerParams(vmem_limit_bytes=_VMEM),
    )(*feat_in)

    head_in = (
        ut, uv, ua,
        proj_t_w, r(proj_t_b),
        r(proj_t_ln_g), r(proj_t_ln_b),
        proj_v_w, r(proj_v_b),
        r(proj_v_ln_g), r(proj_v_ln_b),
        proj_a_w, r(proj_a_b),
        r(proj_a_ln_g), r(proj_a_ln_b),
        priv_t_w, r(priv_t_b), priv_v_w, r(priv_v_b), priv_a_w, r(priv_a_b),
        shared_w, r(shared_b), spd_w, r(spd_b),
        tx_in_w, r(tx_in_b), tx_out_w, r(tx_out_b),
        r(tx_ln1_g), r(tx_ln1_b),
        tx_ff1_w, r(tx_ff1_b), tx_ff2_w, r(tx_ff2_b),
        r(tx_ln2_g), r(tx_ln2_b),
        fusion_w, r(fusion_b),
    )
    E = shared_w.shape[0]
    o, spt, spv, spa, sps = pl.pallas_call(
        functools.partial(_head_kernel, nhead=2,
                          dims=((Ht, Hpt), (Hv, Hpv), (Ha, Hpa))),
        out_shape=(
            jax.ShapeDtypeStruct((B, 3 * E), jnp.float32),
            jax.ShapeDtypeStruct((B, 4), jnp.float32),
            jax.ShapeDtypeStruct((B, 4), jnp.float32),
            jax.ShapeDtypeStruct((B, 4), jnp.float32),
            jax.ShapeDtypeStruct((B, 4), jnp.float32),
        ),
        scratch_shapes=[
            pltpu.VMEM((4 * Hpt, E), jnp.float32),
            pltpu.VMEM((4 * Hpv, E), jnp.float32),
            pltpu.VMEM((4 * Hpa, E), jnp.float32),
        ],
        compiler_params=pltpu.CompilerParams(vmem_limit_bytes=_VMEM),
    )(*head_in)
    aux = {"sp_p_t": spt, "sp_p_v": spv, "sp_p_a": spa, "sp_s": sps}
    return o, aux
